# baseline (speedup 1.0000x reference)
import functools

import jax
import jax.numpy as jnp
from jax import lax
from jax.experimental import pallas as pl
from jax.experimental.pallas import tpu as pltpu
from jax.experimental.pallas import tpu_sc as plsc

VOCAB = 100000
DIM = 64
B = 1024
SENT_COUNT = 10
SENT_LENGTH = 20

N_UR = B * SENT_COUNT * SENT_LENGTH
N_UI = B * SENT_LENGTH
N_ALL = 2 * N_UR + N_UI
PACKED = N_ALL // 2

NC = 2
NS = 16
NW = NC * NS

CHUNK = 128
K = 5
GROUP = K * CHUNK
PGROUP = GROUP // 2

PW_UR = N_UR // NW
PW_UI = N_UI // NW
CH_UR = PW_UR // CHUNK
CH_UI = PW_UI // CHUNK
NCH = 2 * CH_UR + CH_UI
NG_UR = CH_UR // K
NG = NCH // K


def _emb_kernel(ur_idx, ir_idx, ui_idx, table, out,
                idx_v, rows_v, gsem0, gsem1, ssem0, ssem1):
    wid = lax.axis_index("s") * NC + lax.axis_index("c")
    gsems = (gsem0, gsem1)
    ssems = (ssem0, ssem1)

    pltpu.sync_copy(ur_idx.at[pl.ds(wid * CH_UR, CH_UR)],
                    idx_v.at[pl.ds(0, CH_UR)])
    pltpu.sync_copy(ir_idx.at[pl.ds(wid * CH_UR, CH_UR)],
                    idx_v.at[pl.ds(CH_UR, CH_UR)])
    pltpu.sync_copy(ui_idx.at[pl.ds(wid * CH_UI, CH_UI)],
                    idx_v.at[pl.ds(2 * CH_UR, CH_UI)])

    def fire_group(g, p):
        for b in range(K):
            pltpu.async_copy(
                table.at[idx_v.at[g * K + b]],
                rows_v.at[p, pl.ds(b * CHUNK, CHUNK)],
                gsems[p])

    def drain_gathers(p):
        pltpu.make_async_copy(table.at[pl.ds(0, GROUP)],
                              rows_v.at[p], gsems[p]).wait()

    def store_group(p, row_off):
        pltpu.async_copy(
            rows_v.at[p],
            out.at[pl.ds(row_off, GROUP)],
            ssems[p]).wait()

    fire_group(0, 0)
    fire_group(1, 1)

    def make_body(region_base, gbase):
        def body(i, carry):
            s = gbase + 2 * i
            for p in (0, 1):
                g = s + p
                drain_gathers(p)
                store_group(p, region_base + wid * PW_UR
                            + (g - gbase) * GROUP)
                fire_group(g + 2, p)
            return carry
        return body

    lax.fori_loop(0, NG_UR // 2, make_body(0, 0), 0)
    lax.fori_loop(0, NG_UR // 2 - 1, make_body(N_UR, NG_UR), 0)
    for p in (0, 1):
        drain_gathers(p)
        store_group(p, N_UR + wid * PW_UR + (NG_UR - 2 + p) * GROUP)
        if p == 0:
            fire_group(2 * NG_UR, 0)
    drain_gathers(0)
    store_group(0, 2 * N_UR + wid * PW_UI)


def _unpack(x):
    a = x[:, None, :DIM]
    b = x[:, None, DIM:]
    return jnp.concatenate([a, b], axis=1).reshape(2 * x.shape[0], DIM)


def _depad_kernel(ur_in, ir_in, ui_in, ur_out, ir_out, ui_out):
    ur_out[...] = _unpack(ur_in[...])
    ir_out[...] = _unpack(ir_in[...])
    ui_out[...] = _unpack(ui_in[...])


@jax.jit
def _run(ur_flat, ir_flat, ui_flat, word_emb):
    mesh = plsc.VectorSubcoreMesh(core_axis_name="c", subcore_axis_name="s")
    flat = pl.kernel(
        _emb_kernel,
        mesh=mesh,
        out_type=jax.ShapeDtypeStruct((N_ALL, DIM), jnp.float32),
        scratch_types=[
            pltpu.VMEM((NCH, CHUNK), jnp.int32),
            pltpu.VMEM((2, GROUP, DIM), jnp.float32),
            pltpu.SemaphoreType.DMA,
            pltpu.SemaphoreType.DMA,
            pltpu.SemaphoreType.DMA,
            pltpu.SemaphoreType.DMA,
        ],
        compiler_params=pltpu.CompilerParams(use_tc_tiling_on_sc=False),
    )(ur_flat, ir_flat, ui_flat, word_emb)
    packed = flat.reshape(PACKED, 2 * DIM)

    bs = 8
    g = B // bs
    ur_rows = bs * SENT_COUNT * SENT_LENGTH // 2
    ui_rows = bs * SENT_LENGTH // 2
    return pl.pallas_call(
        _depad_kernel,
        grid=(g,),
        in_specs=[
            pl.BlockSpec((ur_rows, 2 * DIM), lambda i: (i, 0)),
            pl.BlockSpec((ur_rows, 2 * DIM), lambda i: (N_UR // 2 // ur_rows + i, 0)),
            pl.BlockSpec((ui_rows, 2 * DIM), lambda i: (N_UR // ui_rows + i, 0)),
        ],
        out_specs=[
            pl.BlockSpec((2 * ur_rows, DIM), lambda i: (i, 0)),
            pl.BlockSpec((2 * ur_rows, DIM), lambda i: (i, 0)),
            pl.BlockSpec((2 * ui_rows, DIM), lambda i: (i, 0)),
        ],
        out_shape=[
            jax.ShapeDtypeStruct((N_UR, DIM), jnp.float32),
            jax.ShapeDtypeStruct((N_UR, DIM), jnp.float32),
            jax.ShapeDtypeStruct((N_UI, DIM), jnp.float32),
        ],
    )(packed, packed, packed)


def kernel(user_reviews, item_reviews, ui_review, word_emb):
    ur = user_reviews.reshape(-1, CHUNK)
    ir = item_reviews.reshape(-1, CHUNK)
    ui = ui_review.reshape(-1, CHUNK)
    out_ur, out_ir, out_ui = _run(ur, ir, ui, word_emb)
    return (
        out_ur.reshape(B, SENT_COUNT * SENT_LENGTH, DIM),
        out_ir.reshape(B, SENT_COUNT * SENT_LENGTH, DIM),
        out_ui.reshape(B, SENT_LENGTH, DIM),
    )


# speedup vs baseline: 1.2833x; 1.2833x over previous
import functools

import jax
import jax.numpy as jnp
from jax import lax
from jax.experimental import pallas as pl
from jax.experimental.pallas import tpu as pltpu
from jax.experimental.pallas import tpu_sc as plsc

VOCAB = 100000
DIM = 64
B = 1024
SENT_COUNT = 10
SENT_LENGTH = 20

N_UR = B * SENT_COUNT * SENT_LENGTH
N_UI = B * SENT_LENGTH

NC = 2
NS = 16
NW = NC * NS

CHUNK = 128
K = 5
GROUP = K * CHUNK

CH_UR = N_UR // NW // CHUNK
CH_UI = N_UI // NW // CHUNK
NCH = 2 * CH_UR + CH_UI
NG_UR = CH_UR // K
NG = NCH // K

PW_UR = N_UR // NW
PW_UI = N_UI // NW


def _emb_kernel(idx_hbm, table,
                out_ur, out_ir, out_ui,
                idx_v, rows_v, gsem0, gsem1, ssem0, ssem1):
    wid = lax.axis_index("s") * NC + lax.axis_index("c")
    gsems = (gsem0, gsem1)
    ssems = (ssem0, ssem1)

    pltpu.sync_copy(idx_hbm.at[pl.ds(wid * CH_UR, CH_UR)],
                    idx_v.at[pl.ds(0, CH_UR)])
    pltpu.sync_copy(idx_hbm.at[pl.ds(NW * CH_UR + wid * CH_UR, CH_UR)],
                    idx_v.at[pl.ds(CH_UR, CH_UR)])
    pltpu.sync_copy(idx_hbm.at[pl.ds(2 * NW * CH_UR + wid * CH_UI, CH_UI)],
                    idx_v.at[pl.ds(2 * CH_UR, CH_UI)])

    def fire_group(g, p):
        for b in range(K):
            pltpu.async_copy(
                table.at[idx_v.at[g * K + b]],
                rows_v.at[p, pl.ds(b * CHUNK, CHUNK)],
                gsems[p])

    def drain_gathers(p):
        pltpu.make_async_copy(table.at[pl.ds(0, GROUP)],
                              rows_v.at[p], gsems[p]).wait()

    fire_group(0, 0)
    fire_group(1, 1)

    def make_body(out_hbm, gbase, guard):
        def body(i, carry):
            s = gbase + 2 * i
            for p in (0, 1):
                g = s + p
                drain_gathers(p)
                st = pltpu.async_copy(
                    rows_v.at[p],
                    out_hbm.at[pl.ds(wid * PW_UR + (g - gbase) * GROUP,
                                     GROUP)],
                    ssems[p])
                st.wait()
                if guard:
                    @pl.when(g + 2 < NG)
                    def _():
                        fire_group(g + 2, p)
                else:
                    fire_group(g + 2, p)
            return carry
        return body

    lax.fori_loop(0, NG_UR // 2, make_body(out_ur, 0, False), 0)
    lax.fori_loop(0, NG_UR // 2, make_body(out_ir, NG_UR, True), 0)
    drain_gathers(0)
    pltpu.sync_copy(rows_v.at[0], out_ui.at[pl.ds(wid * PW_UI, PW_UI)])


@jax.jit
def _run(idx_all, word_emb):
    mesh = plsc.VectorSubcoreMesh(core_axis_name="c", subcore_axis_name="s")
    return pl.kernel(
        _emb_kernel,
        mesh=mesh,
        out_type=[
            jax.ShapeDtypeStruct((N_UR, DIM), jnp.float32),
            jax.ShapeDtypeStruct((N_UR, DIM), jnp.float32),
            jax.ShapeDtypeStruct((N_UI, DIM), jnp.float32),
        ],
        scratch_types=[
            pltpu.VMEM((NCH, CHUNK), jnp.int32),
            pltpu.VMEM((2, GROUP, DIM), jnp.float32),
            pltpu.SemaphoreType.DMA,
            pltpu.SemaphoreType.DMA,
            pltpu.SemaphoreType.DMA,
            pltpu.SemaphoreType.DMA,
        ],
        compiler_params=pltpu.CompilerParams(use_tc_tiling_on_sc=False),
    )(idx_all, word_emb)


def kernel(user_reviews, item_reviews, ui_review, word_emb):
    idx_all = jnp.concatenate([
        user_reviews.reshape(-1, CHUNK),
        item_reviews.reshape(-1, CHUNK),
        ui_review.reshape(-1, CHUNK),
    ])
    out_ur, out_ir, out_ui = _run(idx_all, word_emb)
    return (
        out_ur.reshape(B, SENT_COUNT * SENT_LENGTH, DIM),
        out_ir.reshape(B, SENT_COUNT * SENT_LENGTH, DIM),
        out_ui.reshape(B, SENT_LENGTH, DIM),
    )
